# Initial kernel scaffold; baseline (speedup 1.0000x reference)
import jax
import jax.numpy as jnp
from jax.experimental import pallas as pl


def kernel(x, W_enc, b_enc, W_dec):
    raise NotImplementedError("write your pallas kernel here")


# speedup vs baseline: 9.1236x; 9.1236x over previous
import functools

import jax
import jax.numpy as jnp
from jax.experimental import pallas as pl


def _encode_body(x_ref, w_ref, b_ref, act_ref):
    x = x_ref[...]
    w = w_ref[...]
    act = jax.lax.dot_general(
        x, w, (((1,), (1,)), ((), ())),
        preferred_element_type=jnp.float32)
    act_ref[...] = act + b_ref[...]


def _encode(x, w_enc, b_enc, r_blk, d_blk):
    n, h = x.shape
    dsz = w_enc.shape[0]
    grid = (dsz // d_blk, n // r_blk)
    return pl.pallas_call(
        _encode_body,
        grid=grid,
        in_specs=[
            pl.BlockSpec((r_blk, h), lambda d, r: (r, 0)),
            pl.BlockSpec((d_blk, h), lambda d, r: (d, 0)),
            pl.BlockSpec((1, d_blk), lambda d, r: (0, d)),
        ],
        out_specs=pl.BlockSpec((r_blk, d_blk), lambda d, r: (r, d)),
        out_shape=jax.ShapeDtypeStruct((n, dsz), jnp.float32),
    )(x, w_enc, b_enc.reshape(1, dsz))


def _threshold_body(act_ref, thr_ref, *, k):
    act = act_ref[...]
    bits = jax.lax.bitcast_convert_type(act, jnp.int32)
    key = jnp.where(bits < 0, bits ^ jnp.int32(0x7FFFFFFF), bits)
    rows = act.shape[0]
    t0 = jnp.full((rows, 1), jnp.int32(-2147483648))

    def step(i, t):
        bit = jnp.left_shift(jnp.int32(1), jnp.int32(31) - i)
        cand = t + bit
        cnt = jnp.sum((key >= cand).astype(jnp.int32), axis=1, keepdims=True)
        return jnp.where(cnt >= k, cand, t)

    t = jax.lax.fori_loop(0, 32, step, t0)
    tb = jnp.where(t < 0, t ^ jnp.int32(0x7FFFFFFF), t)
    thr_ref[...] = jax.lax.bitcast_convert_type(tb, jnp.float32)


def _thresholds(act, k, r_blk):
    n, dsz = act.shape
    return pl.pallas_call(
        functools.partial(_threshold_body, k=k),
        grid=(n // r_blk,),
        in_specs=[pl.BlockSpec((r_blk, dsz), lambda r: (r, 0))],
        out_specs=pl.BlockSpec((r_blk, 1), lambda r: (r, 0)),
        out_shape=jax.ShapeDtypeStruct((n, 1), jnp.float32),
    )(act)


def _sparse_decode_body(act_ref, thr_ref, w_ref, sparse_ref, rec_ref):
    d = pl.program_id(1)
    act = act_ref[...]
    sparse = jnp.where(act >= thr_ref[...], act, jnp.float32(0.0))
    sparse_ref[...] = sparse
    part = jax.lax.dot_general(
        sparse, w_ref[...], (((1,), (1,)), ((), ())),
        preferred_element_type=jnp.float32)

    @pl.when(d == 0)
    def _():
        rec_ref[...] = jnp.zeros_like(rec_ref)

    rec_ref[...] += part


def _sparse_decode(act, thr, w_dec, r_blk, d_blk):
    n, dsz = act.shape
    h = w_dec.shape[0]
    grid = (n // r_blk, dsz // d_blk)
    return pl.pallas_call(
        _sparse_decode_body,
        grid=grid,
        in_specs=[
            pl.BlockSpec((r_blk, d_blk), lambda r, d: (r, d)),
            pl.BlockSpec((r_blk, 1), lambda r, d: (r, 0)),
            pl.BlockSpec((h, d_blk), lambda r, d: (0, d)),
        ],
        out_specs=[
            pl.BlockSpec((r_blk, d_blk), lambda r, d: (r, d)),
            pl.BlockSpec((r_blk, h), lambda r, d: (r, 0)),
        ],
        out_shape=[
            jax.ShapeDtypeStruct((n, dsz), jnp.float32),
            jax.ShapeDtypeStruct((n, h), jnp.float32),
        ],
    )(act, thr, w_dec)


def kernel(x, W_enc, b_enc, W_dec, k=64):
    n, h = x.shape
    dsz = W_enc.shape[0]
    r_a = min(256, n)
    d_a = min(2048, dsz)
    act = _encode(x, W_enc, b_enc, r_a, d_a)
    thr = _thresholds(act, k, min(128, n))
    sparse, rec = _sparse_decode(act, thr, W_dec, min(512, n), min(1024, dsz))
    return (rec, sparse)
